# baseline (speedup 1.0000x reference)
import functools

import jax
import jax.numpy as jnp
from jax import lax
from jax.experimental import pallas as pl
from jax.experimental.pallas import tpu as pltpu
from jax.experimental.pallas import tpu_sc as plsc


def _i32(v):
    v &= (1 << 32) - 1
    return v - (1 << 32) if v >= (1 << 31) else v


_P1 = _i32(2654435761)
_P2 = _i32(2246822519)
_P3 = _i32(3266489917)
_INIT = [
    _i32(1 + 2654435761 + 2246822519),
    _i32(1 + 2246822519),
    _i32(1),
    _i32(1 - 2654435761),
]

_NW = 32
_CHUNK = 2048
_L = 16
_TW = 1 << 19


def _shr(v, n):
    return lax.shift_right_logical(v, jnp.full(v.shape, n, v.dtype))


def _rotl(v, n):
    return (v << n) | _shr(v, 32 - n)


def _hash16(w):
    accs = []
    for l in range(4):
        acc = jnp.full((_L,), _INIT[l], jnp.int32)
        for s in range(4):
            acc = acc + w[4 * l + s] * _P2
            acc = _rotl(acc, 13)
            acc = acc * _P1
        accs.append(acc)
    h = (_rotl(accs[0], 1) + _rotl(accs[1], 7)
         + _rotl(accs[2], 12) + _rotl(accs[3], 18))
    h = h + jnp.int32(4)
    h = h ^ _shr(h, 15)
    h = h * _P2
    h = h ^ _shr(h, 13)
    h = h * _P3
    h = h ^ _shr(h, 16)
    return h


def _sc_body(nrows, y_hbm, bs_hbm, out_hbm, tbl_hbm,
             tb8, tb32, yb0, yb1, widx, sbuf, wbuf, obuf, xsem, gsem):
    rows_w = nrows // _NW
    nchunk = rows_w // _CHUNK
    groups = _CHUNK // _L
    tpc = _CHUNK // 128

    cid = lax.axis_index("c")
    sid = lax.axis_index("s")
    wid = sid * 2 + cid
    row0 = wid * rows_w
    tbl0 = cid * _TW

    bytes_per_tile = (_TW * 4) // 16
    stage = tb8.shape[0]
    byte0 = sid * bytes_per_tile

    def conv_pass(p, _):
        off = pl.multiple_of(byte0 + p * stage, 64)
        pltpu.sync_copy(bs_hbm.at[pl.ds(off, stage)], tb8)

        def conv_vec(i, _):
            q = i >> 3
            r = i & 7
            ob = pl.multiple_of(q * 512 + r * 16, 16)
            ow = pl.multiple_of(q * 128 + r * 16, 8)
            tb32[pl.ds(ow, _L)] = plsc.bitcast(tb8[pl.ds(ob, 64)], jnp.int32)
            return ()

        lax.fori_loop(0, stage // 64, conv_vec, (), unroll=False)
        pltpu.sync_copy(tb32, tbl_hbm.at[pl.ds(
            pl.multiple_of(tbl0 + off // 4, 8), stage // 4)])
        return ()

    def conv_all():
        lax.fori_loop(0, bytes_per_tile // stage, conv_pass, (), unroll=False)
        plsc.subcore_barrier()

    def tc_of(c):
        return pl.multiple_of((row0 + c * _CHUNK) // 128, 8)

    def x_copies(c, p):
        tc0 = tc_of(c)
        return (
            pltpu.make_async_copy(y_hbm.at[0, pl.ds(tc0, tpc), :, :],
                                  yb0.at[p], xsem),
            pltpu.make_async_copy(y_hbm.at[1, pl.ds(tc0, tpc), :, :],
                                  yb1.at[p], xsem),
        )

    def x_start(c, p):
        for d in x_copies(c, p):
            d.start()

    def x_wait(c, p):
        for d in x_copies(c, p):
            d.wait()

    def hash_chunk(p):
        def group_body(g, _):
            tcl = g >> 3
            o16 = pl.multiple_of((g & 7) * _L, 16)
            w = [plsc.bitcast((yb0 if j < 8 else yb1)[p, tcl, j & 7,
                                                      pl.ds(o16, _L)],
                              jnp.int32)
                 for j in range(16)]
            h = _hash16(w)
            o = pl.multiple_of(g * _L, 8)
            widx[p, pl.ds(o, _L)] = ((_shr(h, 13) & jnp.int32(-128))
                                     | (_shr(h, 11) & jnp.int32(127))) + tbl0
            sbuf[p, pl.ds(o, _L)] = ((_shr(h, 15) & jnp.int32(24))
                                     | (_shr(h, 8) & jnp.int32(7)))
            return ()

        lax.fori_loop(0, groups, group_body, (), unroll=2)

    def g_descs(p):
        return [
            pltpu.make_async_copy(
                tbl_hbm.at[widx.at[p, pl.ds(r * 128, 128)]],
                wbuf.at[pl.ds(r * 128, 128)],
                gsem,
            )
            for r in range(_CHUNK // 128)
        ]

    def g_fire(p):
        for d in g_descs(p):
            d.start()

    def g_drain(p):
        for d in g_descs(p):
            d.wait()

    def bits_out(c, p):
        def bit_body(g, _):
            o = pl.multiple_of(g * _L, 8)
            wv = wbuf[pl.ds(o, _L)]
            sv = sbuf[p, pl.ds(o, _L)]
            obuf[pl.ds(o, _L)] = lax.shift_right_logical(wv, sv) & jnp.int32(1)
            return ()

        lax.fori_loop(0, groups, bit_body, (), unroll=4)
        base_row = row0 + c * _CHUNK
        pltpu.sync_copy(obuf, out_hbm.at[pl.ds(pl.multiple_of(base_row, 8),
                                               _CHUNK)])

    x_start(0, 0)
    x_start(1, 1)
    conv_all()
    x_wait(0, 0)
    hash_chunk(0)
    x_start(2, 0)
    g_fire(0)

    def pipe_body(k, _):
        p = k & 1
        x_wait(k, p)
        hash_chunk(p)

        @pl.when(k < nchunk - 2)
        def _():
            x_start(k + 2, p)

        g_drain(1 - p)
        bits_out(k - 1, 1 - p)
        g_fire(p)
        return ()

    lax.fori_loop(1, nchunk, pipe_body, (), unroll=False)
    g_drain((nchunk - 1) & 1)
    bits_out(nchunk - 1, (nchunk - 1) & 1)


@jax.jit
def _seen_i32(y, bs):
    nrows = y.shape[1] * 128
    mesh = plsc.VectorSubcoreMesh(core_axis_name="c", subcore_axis_name="s")
    f = pl.kernel(
        functools.partial(_sc_body, nrows),
        out_type=(
            jax.ShapeDtypeStruct((nrows,), jnp.int32),
            jax.ShapeDtypeStruct((2 * _TW,), jnp.int32),
        ),
        mesh=mesh,
        compiler_params=pltpu.CompilerParams(needs_layout_passes=False),
        scratch_types=[
            pltpu.VMEM((16384,), jnp.uint8),
            pltpu.VMEM((4096,), jnp.int32),
            pltpu.VMEM((2, _CHUNK // 128, 8, 128), jnp.float32),
            pltpu.VMEM((2, _CHUNK // 128, 8, 128), jnp.float32),
            pltpu.VMEM((2, _CHUNK), jnp.int32),
            pltpu.VMEM((2, _CHUNK), jnp.int32),
            pltpu.VMEM((_CHUNK,), jnp.int32),
            pltpu.VMEM((_CHUNK,), jnp.int32),
            pltpu.SemaphoreType.DMA,
            pltpu.SemaphoreType.DMA,
        ],
    )
    seen, _ = f(y, bs)
    return seen


def kernel(x, binary_set):
    n = x.shape[0]
    y = jnp.transpose(x.reshape(n // 128, 128, 2, 8), (2, 0, 3, 1))
    return _seen_i32(y, binary_set) != 0


# speedup vs baseline: 1.0346x; 1.0346x over previous
import functools

import jax
import jax.numpy as jnp
from jax import lax
from jax.experimental import pallas as pl
from jax.experimental.pallas import tpu as pltpu
from jax.experimental.pallas import tpu_sc as plsc


def _i32(v):
    v &= (1 << 32) - 1
    return v - (1 << 32) if v >= (1 << 31) else v


_P1 = _i32(2654435761)
_P2 = _i32(2246822519)
_P3 = _i32(3266489917)
_INIT = [
    _i32(1 + 2654435761 + 2246822519),
    _i32(1 + 2246822519),
    _i32(1),
    _i32(1 - 2654435761),
]

_NW = 32
_CHUNK = 2048
_L = 16
_TW = 1 << 19


def _shr(v, n):
    return lax.shift_right_logical(v, jnp.full(v.shape, n, v.dtype))


def _rotl(v, n):
    return (v << n) | _shr(v, 32 - n)


def _hash16(w):
    accs = []
    for l in range(4):
        acc = jnp.full((_L,), _INIT[l], jnp.int32)
        for s in range(4):
            acc = acc + w[4 * l + s] * _P2
            acc = _rotl(acc, 13)
            acc = acc * _P1
        accs.append(acc)
    h = (_rotl(accs[0], 1) + _rotl(accs[1], 7)
         + _rotl(accs[2], 12) + _rotl(accs[3], 18))
    h = h + jnp.int32(4)
    h = h ^ _shr(h, 15)
    h = h * _P2
    h = h ^ _shr(h, 13)
    h = h * _P3
    h = h ^ _shr(h, 16)
    return h


def _sc_body(nrows, y_hbm, bs_hbm, out_hbm, tbl_hbm,
             tb8, tb32, yb0, yb1, widx, sbuf, wbuf, obuf, xsem, gsem):
    rows_w = nrows // _NW
    nchunk = rows_w // _CHUNK
    groups = _CHUNK // _L
    tpc = _CHUNK // 128

    cid = lax.axis_index("c")
    sid = lax.axis_index("s")
    wid = sid * 2 + cid
    row0 = wid * rows_w
    tbl0 = cid * _TW

    bytes_per_tile = (_TW * 4) // 16
    stage = tb8.shape[0]
    byte0 = sid * bytes_per_tile

    def conv_pass(p, _):
        off = pl.multiple_of(byte0 + p * stage, 64)
        pltpu.sync_copy(bs_hbm.at[pl.ds(off, stage)], tb8)

        def conv_vec(i, _):
            q = i >> 3
            r = i & 7
            ob = pl.multiple_of(q * 512 + r * 16, 16)
            ow = pl.multiple_of(q * 128 + r * 16, 8)
            tb32[pl.ds(ow, _L)] = plsc.bitcast(tb8[pl.ds(ob, 64)], jnp.int32)
            return ()

        lax.fori_loop(0, stage // 64, conv_vec, (), unroll=4)
        pltpu.sync_copy(tb32, tbl_hbm.at[pl.ds(
            pl.multiple_of(tbl0 + off // 4, 8), stage // 4)])
        return ()

    def conv_all():
        lax.fori_loop(0, bytes_per_tile // stage, conv_pass, (), unroll=False)
        plsc.subcore_barrier()

    def tc_of(c):
        return pl.multiple_of((row0 + c * _CHUNK) // 128, 8)

    def x_copies(c, p):
        tc0 = tc_of(c)
        return (
            pltpu.make_async_copy(y_hbm.at[0, pl.ds(tc0, tpc), :, :],
                                  yb0.at[p], xsem),
            pltpu.make_async_copy(y_hbm.at[1, pl.ds(tc0, tpc), :, :],
                                  yb1.at[p], xsem),
        )

    def x_start(c, p):
        for d in x_copies(c, p):
            d.start()

    def x_wait(c, p):
        for d in x_copies(c, p):
            d.wait()

    def hash_chunk(p):
        def group_body(g, _):
            tcl = g >> 3
            o16 = pl.multiple_of((g & 7) * _L, 16)
            w = [plsc.bitcast((yb0 if j < 8 else yb1)[p, tcl, j & 7,
                                                      pl.ds(o16, _L)],
                              jnp.int32)
                 for j in range(16)]
            h = _hash16(w)
            o = pl.multiple_of(g * _L, 8)
            widx[p, pl.ds(o, _L)] = ((_shr(h, 13) & jnp.int32(-128))
                                     | (_shr(h, 11) & jnp.int32(127))) + tbl0
            sbuf[p, pl.ds(o, _L)] = ((_shr(h, 15) & jnp.int32(24))
                                     | (_shr(h, 8) & jnp.int32(7)))
            return ()

        lax.fori_loop(0, groups, group_body, (), unroll=False)

    def g_descs(p):
        return [
            pltpu.make_async_copy(
                tbl_hbm.at[widx.at[p, pl.ds(r * 128, 128)]],
                wbuf.at[pl.ds(r * 128, 128)],
                gsem,
            )
            for r in range(_CHUNK // 128)
        ]

    def g_fire(p):
        for d in g_descs(p):
            d.start()

    def g_drain(p):
        for d in g_descs(p):
            d.wait()

    def bits_out(c, p):
        def bit_body(g, _):
            o = pl.multiple_of(g * _L, 8)
            wv = wbuf[pl.ds(o, _L)]
            sv = sbuf[p, pl.ds(o, _L)]
            obuf[pl.ds(o, _L)] = lax.shift_right_logical(wv, sv) & jnp.int32(1)
            return ()

        lax.fori_loop(0, groups, bit_body, (), unroll=4)
        base_row = row0 + c * _CHUNK
        pltpu.sync_copy(obuf, out_hbm.at[pl.ds(pl.multiple_of(base_row, 8),
                                               _CHUNK)])

    x_start(0, 0)
    x_start(1, 1)
    conv_all()
    x_wait(0, 0)
    hash_chunk(0)
    x_start(2, 0)
    g_fire(0)

    def pipe_body(k, _):
        p = k & 1
        x_wait(k, p)
        hash_chunk(p)

        @pl.when(k < nchunk - 2)
        def _():
            x_start(k + 2, p)

        g_drain(1 - p)
        bits_out(k - 1, 1 - p)
        g_fire(p)
        return ()

    lax.fori_loop(1, nchunk, pipe_body, (), unroll=False)
    g_drain((nchunk - 1) & 1)
    bits_out(nchunk - 1, (nchunk - 1) & 1)


@jax.jit
def _seen_i32(y, bs):
    nrows = y.shape[1] * 128
    mesh = plsc.VectorSubcoreMesh(core_axis_name="c", subcore_axis_name="s")
    f = pl.kernel(
        functools.partial(_sc_body, nrows),
        out_type=(
            jax.ShapeDtypeStruct((nrows,), jnp.int32),
            jax.ShapeDtypeStruct((2 * _TW,), jnp.int32),
        ),
        mesh=mesh,
        compiler_params=pltpu.CompilerParams(needs_layout_passes=False),
        scratch_types=[
            pltpu.VMEM((16384,), jnp.uint8),
            pltpu.VMEM((4096,), jnp.int32),
            pltpu.VMEM((2, _CHUNK // 128, 8, 128), jnp.float32),
            pltpu.VMEM((2, _CHUNK // 128, 8, 128), jnp.float32),
            pltpu.VMEM((2, _CHUNK), jnp.int32),
            pltpu.VMEM((2, _CHUNK), jnp.int32),
            pltpu.VMEM((_CHUNK,), jnp.int32),
            pltpu.VMEM((_CHUNK,), jnp.int32),
            pltpu.SemaphoreType.DMA,
            pltpu.SemaphoreType.DMA,
        ],
    )
    seen, _ = f(y, bs)
    return seen


def kernel(x, binary_set):
    n = x.shape[0]
    y = jnp.transpose(x.reshape(n // 128, 128, 2, 8), (2, 0, 3, 1))
    return _seen_i32(y, binary_set) != 0


# speedup vs baseline: 1.0462x; 1.0113x over previous
import functools

import jax
import jax.numpy as jnp
from jax import lax
from jax.experimental import pallas as pl
from jax.experimental.pallas import tpu as pltpu
from jax.experimental.pallas import tpu_sc as plsc


def _i32(v):
    v &= (1 << 32) - 1
    return v - (1 << 32) if v >= (1 << 31) else v


_P1 = _i32(2654435761)
_P2 = _i32(2246822519)
_P3 = _i32(3266489917)
_INIT = [
    _i32(1 + 2654435761 + 2246822519),
    _i32(1 + 2246822519),
    _i32(1),
    _i32(1 - 2654435761),
]

_NW = 32
_CHUNK = 2048
_L = 16
_TW = 1 << 19


def _shr(v, n):
    return lax.shift_right_logical(v, jnp.full(v.shape, n, v.dtype))


def _rotl(v, n):
    return (v << n) | _shr(v, 32 - n)


def _hash16(w):
    accs = []
    for l in range(4):
        acc = jnp.full((_L,), _INIT[l], jnp.int32)
        for s in range(4):
            acc = acc + w[4 * l + s] * _P2
            acc = _rotl(acc, 13)
            acc = acc * _P1
        accs.append(acc)
    h = (_rotl(accs[0], 1) + _rotl(accs[1], 7)
         + _rotl(accs[2], 12) + _rotl(accs[3], 18))
    h = h + jnp.int32(4)
    h = h ^ _shr(h, 15)
    h = h * _P2
    h = h ^ _shr(h, 13)
    h = h * _P3
    h = h ^ _shr(h, 16)
    return h


def _sc_body(nrows, y_hbm, bs_hbm, out_hbm, tbl_hbm,
             tb8, tb32, yb0, yb1, widx, sbuf, wbuf, obuf, xsem, gsem, osem):
    rows_w = nrows // _NW
    nchunk = rows_w // _CHUNK
    groups = _CHUNK // _L
    tpc = _CHUNK // 128

    cid = lax.axis_index("c")
    sid = lax.axis_index("s")
    wid = sid * 2 + cid
    row0 = wid * rows_w
    tbl0 = cid * _TW

    bytes_per_tile = (_TW * 4) // 16
    stage = tb8.shape[0]
    byte0 = sid * bytes_per_tile

    def conv_pass(p, _):
        off = pl.multiple_of(byte0 + p * stage, 64)
        pltpu.sync_copy(bs_hbm.at[pl.ds(off, stage)], tb8)

        def conv_vec(i, _):
            q = i >> 3
            r = i & 7
            ob = pl.multiple_of(q * 512 + r * 16, 16)
            ow = pl.multiple_of(q * 128 + r * 16, 8)
            tb32[pl.ds(ow, _L)] = plsc.bitcast(tb8[pl.ds(ob, 64)], jnp.int32)
            return ()

        lax.fori_loop(0, stage // 64, conv_vec, (), unroll=4)
        pltpu.sync_copy(tb32, tbl_hbm.at[pl.ds(
            pl.multiple_of(tbl0 + off // 4, 8), stage // 4)])
        return ()

    def conv_all():
        lax.fori_loop(0, bytes_per_tile // stage, conv_pass, (), unroll=False)
        plsc.subcore_barrier()

    def tc_of(c):
        return pl.multiple_of((row0 + c * _CHUNK) // 128, 8)

    def x_copies(c, p):
        tc0 = tc_of(c)
        return (
            pltpu.make_async_copy(y_hbm.at[0, pl.ds(tc0, tpc), :, :],
                                  yb0.at[p], xsem),
            pltpu.make_async_copy(y_hbm.at[1, pl.ds(tc0, tpc), :, :],
                                  yb1.at[p], xsem),
        )

    def x_start(c, p):
        for d in x_copies(c, p):
            d.start()

    def x_wait(c, p):
        for d in x_copies(c, p):
            d.wait()

    def hash_chunk(p):
        def group_body(g, _):
            tcl = g >> 3
            o16 = pl.multiple_of((g & 7) * _L, 16)
            w = [plsc.bitcast((yb0 if j < 8 else yb1)[p, tcl, j & 7,
                                                      pl.ds(o16, _L)],
                              jnp.int32)
                 for j in range(16)]
            h = _hash16(w)
            o = pl.multiple_of(g * _L, 8)
            widx[p, pl.ds(o, _L)] = ((_shr(h, 13) & jnp.int32(-128))
                                     | (_shr(h, 11) & jnp.int32(127))) + tbl0
            sbuf[p, pl.ds(o, _L)] = ((_shr(h, 15) & jnp.int32(24))
                                     | (_shr(h, 8) & jnp.int32(7)))
            return ()

        lax.fori_loop(0, groups, group_body, (), unroll=False)

    def g_descs(p):
        return [
            pltpu.make_async_copy(
                tbl_hbm.at[widx.at[p, pl.ds(r * 128, 128)]],
                wbuf.at[pl.ds(r * 128, 128)],
                gsem,
            )
            for r in range(_CHUNK // 128)
        ]

    def g_fire(p):
        for d in g_descs(p):
            d.start()

    def g_drain(p):
        for d in g_descs(p):
            d.wait()

    def out_desc(c, p):
        base_row = row0 + c * _CHUNK
        return pltpu.make_async_copy(
            obuf.at[p],
            out_hbm.at[pl.ds(pl.multiple_of(base_row, 8), _CHUNK)],
            osem)

    def bits_out(c, p):
        def bit_body(g, _):
            o = pl.multiple_of(g * _L, 8)
            wv = wbuf[pl.ds(o, _L)]
            sv = sbuf[p, pl.ds(o, _L)]
            obuf[p, pl.ds(o, _L)] = (lax.shift_right_logical(wv, sv)
                                     & jnp.int32(1))
            return ()

        lax.fori_loop(0, groups, bit_body, (), unroll=4)
        out_desc(c, p).start()

    x_start(0, 0)
    x_start(1, 1)
    conv_all()
    x_wait(0, 0)
    hash_chunk(0)
    x_start(2, 0)
    g_fire(0)

    def pipe_body(k, _):
        p = k & 1
        x_wait(k, p)
        hash_chunk(p)

        @pl.when(k < nchunk - 2)
        def _():
            x_start(k + 2, p)

        g_drain(1 - p)

        @pl.when(k >= 3)
        def _():
            out_desc(k - 3, 1 - p).wait()

        bits_out(k - 1, 1 - p)
        g_fire(p)
        return ()

    lax.fori_loop(1, nchunk, pipe_body, (), unroll=False)
    g_drain((nchunk - 1) & 1)
    out_desc(nchunk - 3, 1).wait()
    bits_out(nchunk - 1, (nchunk - 1) & 1)
    out_desc(nchunk - 2, 0).wait()
    out_desc(nchunk - 1, 1).wait()


@jax.jit
def _seen_i32(y, bs):
    nrows = y.shape[1] * 128
    mesh = plsc.VectorSubcoreMesh(core_axis_name="c", subcore_axis_name="s")
    f = pl.kernel(
        functools.partial(_sc_body, nrows),
        out_type=(
            jax.ShapeDtypeStruct((nrows,), jnp.int32),
            jax.ShapeDtypeStruct((2 * _TW,), jnp.int32),
        ),
        mesh=mesh,
        compiler_params=pltpu.CompilerParams(needs_layout_passes=False),
        scratch_types=[
            pltpu.VMEM((16384,), jnp.uint8),
            pltpu.VMEM((4096,), jnp.int32),
            pltpu.VMEM((2, _CHUNK // 128, 8, 128), jnp.float32),
            pltpu.VMEM((2, _CHUNK // 128, 8, 128), jnp.float32),
            pltpu.VMEM((2, _CHUNK), jnp.int32),
            pltpu.VMEM((2, _CHUNK), jnp.int32),
            pltpu.VMEM((_CHUNK,), jnp.int32),
            pltpu.VMEM((2, _CHUNK), jnp.int32),
            pltpu.SemaphoreType.DMA,
            pltpu.SemaphoreType.DMA,
            pltpu.SemaphoreType.DMA,
        ],
    )
    seen, _ = f(y, bs)
    return seen


def kernel(x, binary_set):
    n = x.shape[0]
    y = jnp.transpose(x.reshape(n // 128, 128, 2, 8), (2, 0, 3, 1))
    return _seen_i32(y, binary_set) != 0
